# baseline (speedup 1.0000x reference)
import functools

import jax
import jax.numpy as jnp
from jax import lax
from jax.experimental import pallas as pl
from jax.experimental.pallas import tpu as pltpu
from jax.experimental.pallas import tpu_sc as plsc

_NC = 2
_NS = 16
_NW = _NC * _NS
_L = 16
_CHUNK = 128

_D = 32


@functools.lru_cache(maxsize=None)
def _build(B):
    assert B % (_NW * _CHUNK) == 0
    bpw = B // _NW
    nchunk = bpw // _CHUNK
    nblk = bpw // _L

    mesh = plsc.VectorSubcoreMesh(core_axis_name="c", subcore_axis_name="s")

    @functools.partial(
        pl.kernel,
        mesh=mesh,
        out_type=jax.ShapeDtypeStruct((B,), jnp.float32),
        compiler_params=pltpu.CompilerParams(
            needs_layout_passes=False, use_tc_tiling_on_sc=False),
        scratch_types=[
            pltpu.VMEM((nchunk, _CHUNK), jnp.int32),
            pltpu.VMEM((nchunk, _CHUNK), jnp.int32),
            pltpu.VMEM((bpw, _D), jnp.float32),
            pltpu.VMEM((bpw, _D), jnp.float32),
            pltpu.VMEM((_D,), jnp.float32),
            pltpu.VMEM((_L,), jnp.float32),
            pltpu.VMEM((bpw,), jnp.float32),
            pltpu.SemaphoreType.DMA,
        ],
    )
    def gmf(uid_hbm, iid_hbm, ut_hbm, it_hbm, w_hbm, b_hbm, out_hbm,
            uidx, iidx, urows, irows, w_v, b_v, acc, sem):
        wid = lax.axis_index("s") * _NC + lax.axis_index("c")
        base = wid * bpw

        pltpu.sync_copy(uid_hbm.at[wid], uidx)
        pltpu.sync_copy(iid_hbm.at[wid], iidx)
        pltpu.sync_copy(w_hbm, w_v)
        pltpu.sync_copy(b_hbm, b_v)

        copies = []
        for j in range(nchunk):
            copies.append(pltpu.async_copy(
                ut_hbm.at[uidx.at[j]], urows.at[pl.ds(j * _CHUNK, _CHUNK)], sem))
            copies.append(pltpu.async_copy(
                it_hbm.at[iidx.at[j]], irows.at[pl.ds(j * _CHUNK, _CHUNK)], sem))

        zero = jnp.zeros((_L,), jnp.float32)
        for t in range(nblk):
            acc[pl.ds(t * _L, _L)] = zero

        for cp in copies:
            cp.wait()

        iota = lax.iota(jnp.int32, _L)

        def dim_body(d, carry):
            col = jnp.full((_L,), d, dtype=jnp.int32)
            wd = plsc.load_gather(w_v, [col])
            for blk in range(nblk):
                rows = blk * _L + iota
                gu = plsc.load_gather(urows, [rows, col])
                gi = plsc.load_gather(irows, [rows, col])
                plsc.addupdate(acc.at[pl.ds(blk * _L, _L)], gu * gi * wd)
            return carry

        lax.fori_loop(0, _D, dim_body, 0)

        bvec = b_v[...]
        for t in range(nblk):
            x = acc[pl.ds(t * _L, _L)] + bvec
            acc[pl.ds(t * _L, _L)] = 1.0 / (1.0 + jnp.exp(-x))
        pltpu.sync_copy(acc, out_hbm.at[pl.ds(base, bpw)])

    return gmf


def kernel(uid, iid, user_table, item_table, W, b):
    B = uid.shape[0]
    gmf = _build(B)
    uid3 = uid.reshape(_NW, -1, _CHUNK)
    iid3 = iid.reshape(_NW, -1, _CHUNK)
    w_flat = W.reshape(-1)
    b_splat = jnp.broadcast_to(b.reshape(()), (_L,)).astype(jnp.float32)
    out = gmf(uid3, iid3, user_table, item_table, w_flat, b_splat)
    return out.reshape(B, 1)
